# baseline (speedup 1.0000x reference)
import functools

import jax
import jax.numpy as jnp
from jax import lax
from jax.experimental import pallas as pl
from jax.experimental.pallas import tpu as pltpu
from jax.experimental.pallas import tpu_sc as plsc

N = 10000
E = 160000
F = 128
ED = 16
DX = 256
NB = 16
NTASK = 64
TED = 64
NCLS = 3

NC, NSUB = 2, 16
NW = NC * NSUB
CHUNK = 128
NCH = 20
EPW = CHUNK * NCH
EPH = NW * EPW
NHALF = 2
EPAD = NHALF * EPH
ACC_ROWS = 10240
ZPS = ACC_ROWS // NSUB
OPS = 624
DUMMY = N

BE = 2048
BN = 1000


def _silu(v):
    return v * (1.0 / (1.0 + jnp.exp(-v)))


def _pack16(hi_f32, lo_f32):
    hb = lax.bitcast_convert_type(hi_f32, jnp.uint32)
    lb = lax.bitcast_convert_type(lo_f32, jnp.uint32)
    hi = (hb + jnp.uint32(0x8000)) & jnp.uint32(0xFFFF0000)
    lo = (lb + jnp.uint32(0x8000)) >> jnp.uint32(16)
    return lax.bitcast_convert_type(hi | lo, jnp.int32)


def _unpack_hi(x_i32):
    xu = lax.bitcast_convert_type(x_i32, jnp.uint32)
    return lax.bitcast_convert_type(xu & jnp.uint32(0xFFFF0000), jnp.float32)


def _unpack_lo(x_i32):
    xu = lax.bitcast_convert_type(x_i32, jnp.uint32)
    return lax.bitcast_convert_type(xu << jnp.uint32(16), jnp.float32)


@functools.lru_cache(maxsize=None)
def _sc_gather_fn():
    mesh = plsc.VectorSubcoreMesh(core_axis_name="c", subcore_axis_name="s")

    def body(tab, ridx, cidx, hr, hc, idx_r, idx_c,
             br0, br1, bc0, bc1, sr0, sr1, sc0, sc1):
        wid = lax.axis_index("c") * NSUB + lax.axis_index("s")
        base = wid * EPW
        pltpu.sync_copy(ridx.at[wid], idx_r)
        pltpu.sync_copy(cidx.at[wid], idx_c)
        pltpu.async_copy(tab.at[idx_r.at[0]], br0, sr0)
        pltpu.async_copy(tab.at[idx_c.at[0]], bc0, sc0)
        pltpu.async_copy(tab.at[idx_r.at[1]], br1, sr1)
        pltpu.async_copy(tab.at[idx_c.at[1]], bc1, sc1)

        def lane(j, idx, buf, sem, dst):
            pltpu.make_async_copy(tab.at[idx.at[j]], buf, sem).wait()
            pltpu.sync_copy(buf, dst.at[pl.ds(base + j * CHUNK, CHUNK)])

            @pl.when(j + 2 < NCH)
            def _():
                pltpu.async_copy(tab.at[idx.at[j + 2]], buf, sem)

        def step(i, carry):
            j = 2 * i
            lane(j, idx_r, br0, sr0, hr)
            lane(j, idx_c, bc0, sc0, hc)
            lane(j + 1, idx_r, br1, sr1, hr)
            lane(j + 1, idx_c, bc1, sc1, hc)
            return carry

        lax.fori_loop(0, NCH // 2, step, 0)

    return pl.kernel(
        body,
        out_type=(jax.ShapeDtypeStruct((EPH, F), jnp.int32),
                  jax.ShapeDtypeStruct((EPH, F), jnp.int32)),
        mesh=mesh,
        scratch_types=[
            pltpu.VMEM((NCH, CHUNK), jnp.int32),
            pltpu.VMEM((NCH, CHUNK), jnp.int32),
            pltpu.VMEM((CHUNK, F), jnp.int32),
            pltpu.VMEM((CHUNK, F), jnp.int32),
            pltpu.VMEM((CHUNK, F), jnp.int32),
            pltpu.VMEM((CHUNK, F), jnp.int32),
            pltpu.SemaphoreType.DMA,
            pltpu.SemaphoreType.DMA,
            pltpu.SemaphoreType.DMA,
            pltpu.SemaphoreType.DMA,
        ],
    )


@functools.lru_cache(maxsize=None)
def _sc_scatter_fn(do_aux):
    mesh = plsc.VectorSubcoreMesh(core_axis_name="c", subcore_axis_name="s")

    def body(m0, m1, aux0, aux1, sidx0, sidx1, z128, aggm, agga,
             idx0, idx1, val0, val1, sem0, sem1, acc):
        c = lax.axis_index("c")
        s = lax.axis_index("s")
        wid = c * NSUB + s
        base = wid * EPW
        pltpu.sync_copy(sidx0.at[wid], idx0)
        pltpu.sync_copy(sidx1.at[wid], idx1)

        def half_loop(src, idx):
            pltpu.async_copy(src.at[pl.ds(base, CHUNK)], val0, sem0)

            def step(i, carry):
                j = 2 * i
                pltpu.async_copy(src.at[pl.ds(base + (j + 1) * CHUNK, CHUNK)],
                                 val1, sem1)
                pltpu.make_async_copy(src.at[pl.ds(base, CHUNK)], val0,
                                      sem0).wait()
                pltpu.sync_copy(val0, acc.at[idx.at[j]], add=True)

                @pl.when(j + 2 < NCH)
                def _():
                    pltpu.async_copy(
                        src.at[pl.ds(base + (j + 2) * CHUNK, CHUNK)],
                        val0, sem0)

                pltpu.make_async_copy(src.at[pl.ds(base, CHUNK)], val1,
                                      sem1).wait()
                pltpu.sync_copy(val1, acc.at[idx.at[j + 1]], add=True)
                return carry

            lax.fori_loop(0, NCH // 2, step, 0)

        def one_phase(srcs, dst):
            pltpu.sync_copy(z128, acc.at[pl.ds(s * ZPS, ZPS)])
            plsc.subcore_barrier()
            for src, idx in srcs:
                half_loop(src, idx)
            plsc.subcore_barrier()

            @pl.when(s == NSUB - 1)
            def _():
                pltpu.sync_copy(
                    acc.at[pl.ds((NSUB - 1) * OPS, N - (NSUB - 1) * OPS)],
                    dst.at[c].at[pl.ds((NSUB - 1) * OPS, N - (NSUB - 1) * OPS)])

            @pl.when(s < NSUB - 1)
            def _():
                pltpu.sync_copy(acc.at[pl.ds(s * OPS, OPS)],
                                dst.at[c].at[pl.ds(s * OPS, OPS)])

            plsc.subcore_barrier()

        one_phase(((m0, idx0), (m1, idx1)), aggm)
        if do_aux:
            one_phase(((aux0, idx0), (aux1, idx1)), agga)

    return pl.kernel(
        body,
        out_type=(jax.ShapeDtypeStruct((NC, N, F), jnp.float32),
                  jax.ShapeDtypeStruct((NC, N, F), jnp.float32)),
        mesh=mesh,
        scratch_types=[
            pltpu.VMEM((NCH, CHUNK), jnp.int32),
            pltpu.VMEM((NCH, CHUNK), jnp.int32),
            pltpu.VMEM((CHUNK, F), jnp.float32),
            pltpu.VMEM((CHUNK, F), jnp.float32),
            pltpu.SemaphoreType.DMA,
            pltpu.SemaphoreType.DMA,
            pltpu.VMEM_SHARED((ACC_ROWS, F), jnp.float32),
        ],
    )


def _edge_body(hr_ref, hc_ref, ea_ref, a1, b1w, e1, r1, bb1, a2, bb2,
               c1, bc1, c2, m_ref, aux_ref):
    hr32 = hr_ref[...]
    hc32 = hc_ref[...]
    d = _unpack_lo(hr32) - _unpack_lo(hc32)
    radial = jnp.sum(d * d, axis=1, keepdims=True)
    z = jnp.dot(_unpack_hi(hr32), a1[...], preferred_element_type=jnp.float32)
    z = z + jnp.dot(_unpack_hi(hc32), b1w[...],
                    preferred_element_type=jnp.float32)
    z = z + jnp.dot(ea_ref[...], e1[...], preferred_element_type=jnp.float32)
    z = z + radial * r1[...] + bb1[...]
    z = _silu(z)
    mm = _silu(jnp.dot(z, a2[...], preferred_element_type=jnp.float32) + bb2[...])
    t = jnp.dot(_silu(jnp.dot(mm, c1[...], preferred_element_type=jnp.float32)
                      + bc1[...]),
                c2[...], preferred_element_type=jnp.float32)
    one3 = (lax.broadcasted_iota(jnp.int32, (BE, F), 1) == 3).astype(jnp.float32)
    m_ref[...] = mm
    aux_ref[...] = d * t + one3


def _tc_edge(hr, hc, ea, half, w):
    full = lambda shp: pl.BlockSpec(shp, lambda i: (0,) * len(shp))
    nblk = EPH // BE
    return pl.pallas_call(
        _edge_body,
        grid=(nblk,),
        in_specs=[
            pl.BlockSpec((BE, F), lambda i: (i, 0)),
            pl.BlockSpec((BE, F), lambda i: (i, 0)),
            pl.BlockSpec((BE, ED), lambda i, h=half: (i + h * nblk, 0)),
            full((F, F)), full((F, F)), full((ED, F)), full((1, F)),
            full((1, F)), full((F, F)), full((1, F)), full((F, F)),
            full((1, F)), full((F, 1)),
        ],
        out_specs=[
            pl.BlockSpec((BE, F), lambda i: (i, 0)),
            pl.BlockSpec((BE, F), lambda i: (i, 0)),
        ],
        out_shape=[
            jax.ShapeDtypeStruct((EPH, F), jnp.float32),
            jax.ShapeDtypeStruct((EPH, F), jnp.float32),
        ],
    )(hr, hc, ea, *w)


def _node_body(hx_ref, h0x_ref, am_ref, aa_ref,
               n1h, n1a, n1n, nb1, n2, nb2, out_ref, tb_ref):
    hx = hx_ref[...]
    h = hx[:, 0:128]
    aggm = am_ref[0] + am_ref[1]
    agga = aa_ref[0] + aa_ref[1]
    cnt = jnp.maximum(agga[:, 3:4], 1.0)
    mask3 = (lax.broadcasted_iota(jnp.int32, (BN, F), 1) < 3).astype(jnp.float32)
    newc = hx[:, 128:256] + (agga / cnt) * mask3
    z = jnp.dot(h, n1h[...], preferred_element_type=jnp.float32)
    z = z + jnp.dot(aggm, n1a[...], preferred_element_type=jnp.float32)
    z = z + jnp.dot(h0x_ref[...][:, 0:128], n1n[...],
                    preferred_element_type=jnp.float32)
    z = _silu(z + nb1[...])
    hn = h + jnp.dot(z, n2[...], preferred_element_type=jnp.float32) + nb2[...]
    out_ref[:, 0:128] = hn
    out_ref[:, 128:256] = newc
    tb_ref[...] = _pack16(hn, newc)


def _tc_node(hx, h0x, am, aa, w):
    full = lambda shp: pl.BlockSpec(shp, lambda i: (0,) * len(shp))
    agg_spec = pl.BlockSpec((NC, BN, F), lambda i: (0, i, 0))
    return pl.pallas_call(
        _node_body,
        grid=(N // BN,),
        in_specs=[
            pl.BlockSpec((BN, DX), lambda i: (i, 0)),
            pl.BlockSpec((BN, DX), lambda i: (i, 0)),
            agg_spec, agg_spec,
            full((F, F)), full((F, F)), full((F, F)), full((1, F)),
            full((F, F)), full((1, F)),
        ],
        out_specs=[pl.BlockSpec((BN, DX), lambda i: (i, 0)),
                   pl.BlockSpec((BN, F), lambda i: (i, 0))],
        out_shape=[jax.ShapeDtypeStruct((N, DX), jnp.float32),
                   jax.ShapeDtypeStruct((N, F), jnp.int32)],
    )(hx, h0x, am, aa, *w)


def _node_pool_body(hx_ref, am_ref, n1h, n1a, nb1, n2, nb2,
                    out_ref):
    h = hx_ref[...][:, 0:128]
    aggm = am_ref[0] + am_ref[1]
    z = jnp.dot(h, n1h[...], preferred_element_type=jnp.float32)
    z = z + jnp.dot(aggm, n1a[...], preferred_element_type=jnp.float32)
    z = _silu(z + nb1[...])
    out_ref[...] = h + jnp.dot(z, n2[...], preferred_element_type=jnp.float32) \
        + nb2[...]


def _tc_node_pool(hx, am, w):
    full = lambda shp: pl.BlockSpec(shp, lambda i: (0,) * len(shp))
    agg_spec = pl.BlockSpec((NC, BN, F), lambda i: (0, i, 0))
    return pl.pallas_call(
        _node_pool_body,
        grid=(N // BN,),
        in_specs=[
            pl.BlockSpec((BN, DX), lambda i: (i, 0)),
            agg_spec,
            full((F, F)), full((F, F)), full((1, F)), full((F, F)),
            full((1, F)),
        ],
        out_specs=pl.BlockSpec((BN, F), lambda i: (i, 0)),
        out_shape=jax.ShapeDtypeStruct((N, F), jnp.float32),
    )(hx, am, *w)


def _embed_body(hx_ref, wt, bb, out_ref, tb_ref):
    hx = hx_ref[...]
    he = jnp.dot(hx[:, 0:128], wt[...],
                 preferred_element_type=jnp.float32) + bb[...]
    out_ref[:, 0:128] = he
    out_ref[:, 128:256] = hx[:, 128:256]
    tb_ref[...] = _pack16(he, hx[:, 128:256])


def _tc_embed(hx, wt, bb):
    full = lambda shp: pl.BlockSpec(shp, lambda i: (0,) * len(shp))
    return pl.pallas_call(
        _embed_body,
        grid=(N // BN,),
        in_specs=[
            pl.BlockSpec((BN, DX), lambda i: (i, 0)),
            full((F, F)), full((1, F)),
        ],
        out_specs=[pl.BlockSpec((BN, DX), lambda i: (i, 0)),
                   pl.BlockSpec((BN, F), lambda i: (i, 0))],
        out_shape=[jax.ShapeDtypeStruct((N, DX), jnp.float32),
                   jax.ShapeDtypeStruct((N, F), jnp.int32)],
    )(hx, wt, bb)


def _head_body(hf_ref, batch_ref, ti_ref, te_ref,
               m0h, m0t, hb0, m1, hb1, m2, hb2, m3, hb3, out_ref):
    hf = hf_ref[...]
    oneh = (lax.broadcasted_iota(jnp.int32, (NB, N), 0)
            == batch_ref[...]).astype(jnp.float32)
    sums = jnp.dot(oneh, hf, preferred_element_type=jnp.float32)
    cnt = jnp.maximum(jnp.sum(oneh, axis=1, keepdims=True), 1.0)
    p = sums / cnt
    teh = (lax.broadcasted_iota(jnp.int32, (NB, NTASK), 1)
           == ti_ref[...]).astype(jnp.float32)
    te = jnp.dot(teh, te_ref[...], preferred_element_type=jnp.float32)
    y = jnp.maximum(jnp.dot(p, m0h[...], preferred_element_type=jnp.float32)
                    + jnp.dot(te, m0t[...], preferred_element_type=jnp.float32)
                    + hb0[...], 0.0)
    y = jnp.maximum(jnp.dot(y, m1[...], preferred_element_type=jnp.float32)
                    + hb1[...], 0.0)
    y = jnp.maximum(jnp.dot(y, m2[...], preferred_element_type=jnp.float32)
                    + hb2[...], 0.0)
    out_ref[...] = jnp.dot(y, m3[...], preferred_element_type=jnp.float32) \
        + hb3[...]


def _tc_head(hf, batch_row, ti, te, w):
    full = lambda shp: pl.BlockSpec(shp, lambda: (0,) * len(shp))
    return pl.pallas_call(
        _head_body,
        in_specs=[
            full((N, F)), full((1, N)), full((NB, 1)), full((NTASK, TED)),
            full((F, F)), full((TED, F)), full((1, F)),
            full((F, F)), full((1, F)),
            full((F, F)), full((1, F)),
            full((F, NCLS)), full((1, NCLS)),
        ],
        out_specs=full((NB, NCLS)),
        out_shape=jax.ShapeDtypeStruct((NB, NCLS), jnp.float32),
    )(hf, batch_row, ti, te, *w)


def _edge_w(p):
    w1 = p["edge1"]["W"]
    return (w1[:, 0:128].T, w1[:, 128:256].T, w1[:, 257:273].T,
            w1[:, 256:257].T, p["edge1"]["b"][None],
            p["edge2"]["W"].T, p["edge2"]["b"][None],
            p["coord1"]["W"].T, p["coord1"]["b"][None],
            p["coord2"]["W"].T)


def _node_w(p):
    wn = p["node1"]["W"]
    return (wn[:, 0:128].T, wn[:, 128:256].T, wn[:, 256:384].T,
            p["node1"]["b"][None], p["node2"]["W"].T, p["node2"]["b"][None])


def _node_pool_w(p):
    wn = p["node1"]["W"]
    return (wn[:, 0:128].T, wn[:, 128:256].T,
            p["node1"]["b"][None], p["node2"]["W"].T, p["node2"]["b"][None])


def _head_w(params):
    w0 = params["mlp0"]["W"]
    return (w0[:, 0:128].T, w0[:, 128:192].T, params["mlp0"]["b"][None],
            params["mlp1"]["W"].T, params["mlp1"]["b"][None],
            params["mlp2"]["W"].T, params["mlp2"]["b"][None],
            params["mlp3"]["W"].T, params["mlp3"]["b"][None])


def kernel(h, x, edge_index, edge_attr, batch, tasks_indices, params):
    f32 = jnp.float32
    row = edge_index[0].astype(jnp.int32)
    col = edge_index[1].astype(jnp.int32)
    npad = EPAD - E
    rowp = jnp.concatenate([row, jnp.zeros((npad,), jnp.int32)])
    colp = jnp.concatenate([col, jnp.zeros((npad,), jnp.int32)])
    dummy = DUMMY + (jnp.arange(npad, dtype=jnp.int32) % (ACC_ROWS - N))
    rowsp = jnp.concatenate([row, dummy])
    rowg = rowp.reshape(NHALF, NW, NCH, CHUNK)
    colg = colp.reshape(NHALF, NW, NCH, CHUNK)
    rows = rowsp.reshape(NHALF, NW, NCH, CHUNK)
    ea = jnp.zeros((EPAD, ED), f32).at[:E].set(edge_attr.astype(f32))
    z128 = jnp.zeros((ZPS, F), f32)

    xpad = jnp.zeros((N, F), f32).at[:, 0:3].set(x.astype(f32))
    hx = jnp.concatenate([h.astype(f32), xpad], axis=1)

    gather = _sc_gather_fn()

    def egcl(hx_cur, tb32, h0x, p, pool):
        scatter = _sc_scatter_fn(not pool)
        ew = _edge_w(p)
        ms, auxs = [], []
        for hh in range(NHALF):
            hr, hc = gather(tb32, rowg[hh], colg[hh])
            m, aux = _tc_edge(hr, hc, ea, hh, ew)
            ms.append(m)
            auxs.append(aux)
        aggm, agga = scatter(ms[0], ms[1], auxs[0], auxs[1],
                             rows[0], rows[1], z128)
        if pool:
            return _tc_node_pool(hx_cur, aggm, _node_pool_w(p))
        return _tc_node(hx_cur, h0x, aggm, agga, _node_w(p))

    tb32 = None
    for ep in params["egnns"]:
        h0x = hx
        hx, tb32 = _tc_embed(hx, ep["embed"]["W"].T, ep["embed"]["b"][None])
        for lp in ep["layers"]:
            hx, tb32 = egcl(hx, tb32, h0x, lp, pool=False)

    hfin = egcl(hx, tb32, None, params["pool_egcl"], pool=True)

    batch_row = batch.astype(jnp.int32).reshape(1, N)
    ti = tasks_indices[:, 1].astype(jnp.int32).reshape(NB, 1)
    return _tc_head(hfin, batch_row, ti, params["tasks_embed"].astype(f32),
                    _head_w(params))


# speedup vs baseline: 1.8284x; 1.8284x over previous
import functools

import jax
import jax.numpy as jnp
from jax import lax
from jax.experimental import pallas as pl
from jax.experimental.pallas import tpu as pltpu
from jax.experimental.pallas import tpu_sc as plsc

N = 10000
E = 160000
F = 128
ED = 16
DX = 256
NB = 16
NTASK = 64
TED = 64
NCLS = 3

NC, NSUB = 2, 16
NW = NC * NSUB
CHUNK = 128
NCH = 20
EPW = CHUNK * NCH
EPH = NW * EPW
NHALF = 2
EPAD = NHALF * EPH
ACC_ROWS = 10240
ZPS = ACC_ROWS // NSUB
OPS = 624
DUMMY = N

BE = 2048
BN = 1000


def _silu(v):
    return v * (1.0 / (1.0 + jnp.exp(-v)))


def _pack16(hi_f32, lo_f32):
    hb = lax.bitcast_convert_type(hi_f32, jnp.uint32)
    lb = lax.bitcast_convert_type(lo_f32, jnp.uint32)
    hi = (hb + jnp.uint32(0x8000)) & jnp.uint32(0xFFFF0000)
    lo = (lb + jnp.uint32(0x8000)) >> jnp.uint32(16)
    return lax.bitcast_convert_type(hi | lo, jnp.int32)


def _unpack_hi(x_i32):
    xu = lax.bitcast_convert_type(x_i32, jnp.uint32)
    return lax.bitcast_convert_type(xu & jnp.uint32(0xFFFF0000), jnp.float32)


def _unpack_lo(x_i32):
    xu = lax.bitcast_convert_type(x_i32, jnp.uint32)
    return lax.bitcast_convert_type(xu << jnp.uint32(16), jnp.float32)


@functools.lru_cache(maxsize=None)
def _sc_gather_fn():
    mesh = plsc.VectorSubcoreMesh(core_axis_name="c", subcore_axis_name="s")

    def body(tab, ridx, cidx, hr, hc, idx_r, idx_c,
             br0, br1, bc0, bc1, sr0, sr1, sc0, sc1):
        wid = lax.axis_index("c") * NSUB + lax.axis_index("s")
        base = wid * EPW
        pltpu.sync_copy(ridx.at[wid], idx_r)
        pltpu.sync_copy(cidx.at[wid], idx_c)
        pltpu.async_copy(tab.at[idx_r.at[0]], br0, sr0)
        pltpu.async_copy(tab.at[idx_c.at[0]], bc0, sc0)
        pltpu.async_copy(tab.at[idx_r.at[1]], br1, sr1)
        pltpu.async_copy(tab.at[idx_c.at[1]], bc1, sc1)

        def lane(j, idx, buf, sem, dst):
            pltpu.make_async_copy(tab.at[idx.at[j]], buf, sem).wait()
            pltpu.sync_copy(buf, dst.at[pl.ds(base + j * CHUNK, CHUNK)])

            @pl.when(j + 2 < NCH)
            def _():
                pltpu.async_copy(tab.at[idx.at[j + 2]], buf, sem)

        def step(i, carry):
            j = 2 * i
            lane(j, idx_r, br0, sr0, hr)
            lane(j, idx_c, bc0, sc0, hc)
            lane(j + 1, idx_r, br1, sr1, hr)
            lane(j + 1, idx_c, bc1, sc1, hc)
            return carry

        lax.fori_loop(0, NCH // 2, step, 0)

    return pl.kernel(
        body,
        out_type=(jax.ShapeDtypeStruct((EPH, F), jnp.int32),
                  jax.ShapeDtypeStruct((EPH, F), jnp.int32)),
        mesh=mesh,
        scratch_types=[
            pltpu.VMEM((NCH, CHUNK), jnp.int32),
            pltpu.VMEM((NCH, CHUNK), jnp.int32),
            pltpu.VMEM((CHUNK, F), jnp.int32),
            pltpu.VMEM((CHUNK, F), jnp.int32),
            pltpu.VMEM((CHUNK, F), jnp.int32),
            pltpu.VMEM((CHUNK, F), jnp.int32),
            pltpu.SemaphoreType.DMA,
            pltpu.SemaphoreType.DMA,
            pltpu.SemaphoreType.DMA,
            pltpu.SemaphoreType.DMA,
        ],
    )


@functools.lru_cache(maxsize=None)
def _sc_scatter_fn(do_aux):
    mesh = plsc.VectorSubcoreMesh(core_axis_name="c", subcore_axis_name="s")

    def body(m0, m1, aux0, aux1, sidx0, sidx1, z128, aggm, agga,
             idx0, idx1, val0, val1, sem0, sem1, acc):
        c = lax.axis_index("c")
        s = lax.axis_index("s")
        wid = c * NSUB + s
        base = wid * EPW
        pltpu.sync_copy(sidx0.at[wid], idx0)
        pltpu.sync_copy(sidx1.at[wid], idx1)

        def half_loop(src, idx):
            pltpu.async_copy(src.at[pl.ds(base, CHUNK)], val0, sem0)

            def step(i, carry):
                j = 2 * i
                pltpu.async_copy(src.at[pl.ds(base + (j + 1) * CHUNK, CHUNK)],
                                 val1, sem1)
                pltpu.make_async_copy(src.at[pl.ds(base, CHUNK)], val0,
                                      sem0).wait()
                pltpu.sync_copy(val0, acc.at[idx.at[j]], add=True)

                @pl.when(j + 2 < NCH)
                def _():
                    pltpu.async_copy(
                        src.at[pl.ds(base + (j + 2) * CHUNK, CHUNK)],
                        val0, sem0)

                pltpu.make_async_copy(src.at[pl.ds(base, CHUNK)], val1,
                                      sem1).wait()
                pltpu.sync_copy(val1, acc.at[idx.at[j + 1]], add=True)
                return carry

            lax.fori_loop(0, NCH // 2, step, 0)

        def one_phase(srcs, dst):
            pltpu.sync_copy(z128, acc.at[pl.ds(s * ZPS, ZPS)])
            plsc.subcore_barrier()
            for src, idx in srcs:
                half_loop(src, idx)
            plsc.subcore_barrier()

            @pl.when(s == NSUB - 1)
            def _():
                pltpu.sync_copy(
                    acc.at[pl.ds((NSUB - 1) * OPS, N - (NSUB - 1) * OPS)],
                    dst.at[c].at[pl.ds((NSUB - 1) * OPS, N - (NSUB - 1) * OPS)])

            @pl.when(s < NSUB - 1)
            def _():
                pltpu.sync_copy(acc.at[pl.ds(s * OPS, OPS)],
                                dst.at[c].at[pl.ds(s * OPS, OPS)])

            plsc.subcore_barrier()

        one_phase(((m0, idx0), (m1, idx1)), aggm)
        if do_aux:
            one_phase(((aux0, idx0), (aux1, idx1)), agga)

    return pl.kernel(
        body,
        out_type=(jax.ShapeDtypeStruct((NC, N, F), jnp.float32),
                  jax.ShapeDtypeStruct((NC, N, F), jnp.float32)),
        mesh=mesh,
        scratch_types=[
            pltpu.VMEM((NCH, CHUNK), jnp.int32),
            pltpu.VMEM((NCH, CHUNK), jnp.int32),
            pltpu.VMEM((CHUNK, F), jnp.float32),
            pltpu.VMEM((CHUNK, F), jnp.float32),
            pltpu.SemaphoreType.DMA,
            pltpu.SemaphoreType.DMA,
            pltpu.VMEM_SHARED((ACC_ROWS, F), jnp.float32),
        ],
    )


def _edge_body(half_off, hr_ref, hc_ref, ea_ref, a1, b1w, e1, r1, bb1,
               a2, bb2, c1, bc1, c2, m_ref, aux_ref):
    hr32 = hr_ref[...]
    hc32 = hc_ref[...]
    d = _unpack_lo(hr32) - _unpack_lo(hc32)
    radial = jnp.sum(d * d, axis=1, keepdims=True)
    z = jnp.dot(_unpack_hi(hr32), a1[...], preferred_element_type=jnp.float32)
    z = z + jnp.dot(_unpack_hi(hc32), b1w[...],
                    preferred_element_type=jnp.float32)
    z = z + jnp.dot(ea_ref[...], e1[...], preferred_element_type=jnp.float32)
    z = z + radial * r1[...] + bb1[...]
    z = _silu(z)
    mm = _silu(jnp.dot(z, a2[...], preferred_element_type=jnp.float32) + bb2[...])
    t = jnp.dot(_silu(jnp.dot(mm, c1[...], preferred_element_type=jnp.float32)
                      + bc1[...]),
                c2[...], preferred_element_type=jnp.float32)
    one3 = (lax.broadcasted_iota(jnp.int32, (BE, F), 1) == 3).astype(jnp.float32)
    gid = (pl.program_id(0) * BE + half_off
           + lax.broadcasted_iota(jnp.int32, (BE, 1), 0))
    live = (gid < E).astype(jnp.float32)
    m_ref[...] = mm * live
    aux_ref[...] = (d * t + one3) * live


def _tc_edge(hr, hc, ea, half, w):
    full = lambda shp: pl.BlockSpec(shp, lambda i: (0,) * len(shp))
    nblk = EPH // BE
    return pl.pallas_call(
        functools.partial(_edge_body, half * EPH),
        grid=(nblk,),
        in_specs=[
            pl.BlockSpec((BE, F), lambda i: (i, 0)),
            pl.BlockSpec((BE, F), lambda i: (i, 0)),
            pl.BlockSpec((BE, ED), lambda i, h=half: (i + h * nblk, 0)),
            full((F, F)), full((F, F)), full((ED, F)), full((1, F)),
            full((1, F)), full((F, F)), full((1, F)), full((F, F)),
            full((1, F)), full((F, 1)),
        ],
        out_specs=[
            pl.BlockSpec((BE, F), lambda i: (i, 0)),
            pl.BlockSpec((BE, F), lambda i: (i, 0)),
        ],
        out_shape=[
            jax.ShapeDtypeStruct((EPH, F), jnp.float32),
            jax.ShapeDtypeStruct((EPH, F), jnp.float32),
        ],
    )(hr, hc, ea, *w)


def _node_body(hx_ref, h0x_ref, am_ref, aa_ref,
               n1h, n1a, n1n, nb1, n2, nb2, out_ref, tb_ref):
    hx = hx_ref[...]
    h = hx[:, 0:128]
    aggm = am_ref[0] + am_ref[1]
    agga = aa_ref[0] + aa_ref[1]
    cnt = jnp.maximum(agga[:, 3:4], 1.0)
    mask3 = (lax.broadcasted_iota(jnp.int32, (BN, F), 1) < 3).astype(jnp.float32)
    newc = hx[:, 128:256] + (agga / cnt) * mask3
    z = jnp.dot(h, n1h[...], preferred_element_type=jnp.float32)
    z = z + jnp.dot(aggm, n1a[...], preferred_element_type=jnp.float32)
    z = z + jnp.dot(h0x_ref[...][:, 0:128], n1n[...],
                    preferred_element_type=jnp.float32)
    z = _silu(z + nb1[...])
    hn = h + jnp.dot(z, n2[...], preferred_element_type=jnp.float32) + nb2[...]
    out_ref[:, 0:128] = hn
    out_ref[:, 128:256] = newc
    tb_ref[...] = _pack16(hn, newc)


def _tc_node(hx, h0x, am, aa, w):
    full = lambda shp: pl.BlockSpec(shp, lambda i: (0,) * len(shp))
    agg_spec = pl.BlockSpec((NC, BN, F), lambda i: (0, i, 0))
    return pl.pallas_call(
        _node_body,
        grid=(N // BN,),
        in_specs=[
            pl.BlockSpec((BN, DX), lambda i: (i, 0)),
            pl.BlockSpec((BN, DX), lambda i: (i, 0)),
            agg_spec, agg_spec,
            full((F, F)), full((F, F)), full((F, F)), full((1, F)),
            full((F, F)), full((1, F)),
        ],
        out_specs=[pl.BlockSpec((BN, DX), lambda i: (i, 0)),
                   pl.BlockSpec((BN, F), lambda i: (i, 0))],
        out_shape=[jax.ShapeDtypeStruct((N, DX), jnp.float32),
                   jax.ShapeDtypeStruct((N, F), jnp.int32)],
    )(hx, h0x, am, aa, *w)


def _node_pool_body(hx_ref, am_ref, n1h, n1a, nb1, n2, nb2,
                    out_ref):
    h = hx_ref[...][:, 0:128]
    aggm = am_ref[0] + am_ref[1]
    z = jnp.dot(h, n1h[...], preferred_element_type=jnp.float32)
    z = z + jnp.dot(aggm, n1a[...], preferred_element_type=jnp.float32)
    z = _silu(z + nb1[...])
    out_ref[...] = h + jnp.dot(z, n2[...], preferred_element_type=jnp.float32) \
        + nb2[...]


def _tc_node_pool(hx, am, w):
    full = lambda shp: pl.BlockSpec(shp, lambda i: (0,) * len(shp))
    agg_spec = pl.BlockSpec((NC, BN, F), lambda i: (0, i, 0))
    return pl.pallas_call(
        _node_pool_body,
        grid=(N // BN,),
        in_specs=[
            pl.BlockSpec((BN, DX), lambda i: (i, 0)),
            agg_spec,
            full((F, F)), full((F, F)), full((1, F)), full((F, F)),
            full((1, F)),
        ],
        out_specs=pl.BlockSpec((BN, F), lambda i: (i, 0)),
        out_shape=jax.ShapeDtypeStruct((N, F), jnp.float32),
    )(hx, am, *w)


def _embed_body(hx_ref, wt, bb, out_ref, tb_ref):
    hx = hx_ref[...]
    he = jnp.dot(hx[:, 0:128], wt[...],
                 preferred_element_type=jnp.float32) + bb[...]
    out_ref[:, 0:128] = he
    out_ref[:, 128:256] = hx[:, 128:256]
    tb_ref[...] = _pack16(he, hx[:, 128:256])


def _tc_embed(hx, wt, bb):
    full = lambda shp: pl.BlockSpec(shp, lambda i: (0,) * len(shp))
    return pl.pallas_call(
        _embed_body,
        grid=(N // BN,),
        in_specs=[
            pl.BlockSpec((BN, DX), lambda i: (i, 0)),
            full((F, F)), full((1, F)),
        ],
        out_specs=[pl.BlockSpec((BN, DX), lambda i: (i, 0)),
                   pl.BlockSpec((BN, F), lambda i: (i, 0))],
        out_shape=[jax.ShapeDtypeStruct((N, DX), jnp.float32),
                   jax.ShapeDtypeStruct((N, F), jnp.int32)],
    )(hx, wt, bb)


def _head_body(hf_ref, batch_ref, ti_ref, te_ref,
               m0h, m0t, hb0, m1, hb1, m2, hb2, m3, hb3, out_ref):
    hf = hf_ref[...]
    oneh = (lax.broadcasted_iota(jnp.int32, (NB, N), 0)
            == batch_ref[...]).astype(jnp.float32)
    sums = jnp.dot(oneh, hf, preferred_element_type=jnp.float32)
    cnt = jnp.maximum(jnp.sum(oneh, axis=1, keepdims=True), 1.0)
    p = sums / cnt
    teh = (lax.broadcasted_iota(jnp.int32, (NB, NTASK), 1)
           == ti_ref[...]).astype(jnp.float32)
    te = jnp.dot(teh, te_ref[...], preferred_element_type=jnp.float32)
    y = jnp.maximum(jnp.dot(p, m0h[...], preferred_element_type=jnp.float32)
                    + jnp.dot(te, m0t[...], preferred_element_type=jnp.float32)
                    + hb0[...], 0.0)
    y = jnp.maximum(jnp.dot(y, m1[...], preferred_element_type=jnp.float32)
                    + hb1[...], 0.0)
    y = jnp.maximum(jnp.dot(y, m2[...], preferred_element_type=jnp.float32)
                    + hb2[...], 0.0)
    out_ref[...] = jnp.dot(y, m3[...], preferred_element_type=jnp.float32) \
        + hb3[...]


def _tc_head(hf, batch_row, ti, te, w):
    full = lambda shp: pl.BlockSpec(shp, lambda: (0,) * len(shp))
    return pl.pallas_call(
        _head_body,
        in_specs=[
            full((N, F)), full((1, N)), full((NB, 1)), full((NTASK, TED)),
            full((F, F)), full((TED, F)), full((1, F)),
            full((F, F)), full((1, F)),
            full((F, F)), full((1, F)),
            full((F, NCLS)), full((1, NCLS)),
        ],
        out_specs=full((NB, NCLS)),
        out_shape=jax.ShapeDtypeStruct((NB, NCLS), jnp.float32),
    )(hf, batch_row, ti, te, *w)


def _edge_w(p):
    w1 = p["edge1"]["W"]
    return (w1[:, 0:128].T, w1[:, 128:256].T, w1[:, 257:273].T,
            w1[:, 256:257].T, p["edge1"]["b"][None],
            p["edge2"]["W"].T, p["edge2"]["b"][None],
            p["coord1"]["W"].T, p["coord1"]["b"][None],
            p["coord2"]["W"].T)


def _node_w(p):
    wn = p["node1"]["W"]
    return (wn[:, 0:128].T, wn[:, 128:256].T, wn[:, 256:384].T,
            p["node1"]["b"][None], p["node2"]["W"].T, p["node2"]["b"][None])


def _node_pool_w(p):
    wn = p["node1"]["W"]
    return (wn[:, 0:128].T, wn[:, 128:256].T,
            p["node1"]["b"][None], p["node2"]["W"].T, p["node2"]["b"][None])


def _head_w(params):
    w0 = params["mlp0"]["W"]
    return (w0[:, 0:128].T, w0[:, 128:192].T, params["mlp0"]["b"][None],
            params["mlp1"]["W"].T, params["mlp1"]["b"][None],
            params["mlp2"]["W"].T, params["mlp2"]["b"][None],
            params["mlp3"]["W"].T, params["mlp3"]["b"][None])


def kernel(h, x, edge_index, edge_attr, batch, tasks_indices, params):
    f32 = jnp.float32
    row = edge_index[0].astype(jnp.int32)
    col = edge_index[1].astype(jnp.int32)
    npad = EPAD - E
    rowp = jnp.concatenate([row, row[:npad]])
    colp = jnp.concatenate([col, col[:npad]])
    rowg = rowp.reshape(NHALF, NW, NCH, CHUNK)
    colg = colp.reshape(NHALF, NW, NCH, CHUNK)
    rows = rowg
    ea = jnp.zeros((EPAD, ED), f32).at[:E].set(edge_attr.astype(f32))
    z128 = jnp.zeros((ZPS, F), f32)

    xpad = jnp.zeros((N, F), f32).at[:, 0:3].set(x.astype(f32))
    hx = jnp.concatenate([h.astype(f32), xpad], axis=1)

    gather = _sc_gather_fn()

    def egcl(hx_cur, tb32, h0x, p, pool):
        scatter = _sc_scatter_fn(not pool)
        ew = _edge_w(p)
        ms, auxs = [], []
        for hh in range(NHALF):
            hr, hc = gather(tb32, rowg[hh], colg[hh])
            m, aux = _tc_edge(hr, hc, ea, hh, ew)
            ms.append(m)
            auxs.append(aux)
        aggm, agga = scatter(ms[0], ms[1], auxs[0], auxs[1],
                             rows[0], rows[1], z128)
        if pool:
            return _tc_node_pool(hx_cur, aggm, _node_pool_w(p))
        return _tc_node(hx_cur, h0x, aggm, agga, _node_w(p))

    tb32 = None
    for ep in params["egnns"]:
        h0x = hx
        hx, tb32 = _tc_embed(hx, ep["embed"]["W"].T, ep["embed"]["b"][None])
        for lp in ep["layers"]:
            hx, tb32 = egcl(hx, tb32, h0x, lp, pool=False)

    hfin = egcl(hx, tb32, None, params["pool_egcl"], pool=True)

    batch_row = batch.astype(jnp.int32).reshape(1, N)
    ti = tasks_indices[:, 1].astype(jnp.int32).reshape(NB, 1)
    return _tc_head(hfin, batch_row, ti, params["tasks_embed"].astype(f32),
                    _head_w(params))


# speedup vs baseline: 1.8552x; 1.0147x over previous
import functools

import jax
import jax.numpy as jnp
from jax import lax
from jax.experimental import pallas as pl
from jax.experimental.pallas import tpu as pltpu
from jax.experimental.pallas import tpu_sc as plsc

N = 10000
E = 160000
F = 128
ED = 16
DX = 256
NB = 16
NTASK = 64
TED = 64
NCLS = 3

NC, NSUB = 2, 16
NW = NC * NSUB
CHUNK = 128
NCH = 20
EPW = CHUNK * NCH
EPH = NW * EPW
NHALF = 2
EPAD = NHALF * EPH
ACC_ROWS = 10240
ZPS = ACC_ROWS // NSUB
OPS = 624
DUMMY = N

BE = 2048
BN = 1000


def _silu(v):
    return v * (1.0 / (1.0 + jnp.exp(-v)))


def _pack16(hi_f32, lo_f32):
    hb = lax.bitcast_convert_type(hi_f32, jnp.uint32)
    lb = lax.bitcast_convert_type(lo_f32, jnp.uint32)
    hi = (hb + jnp.uint32(0x8000)) & jnp.uint32(0xFFFF0000)
    lo = (lb + jnp.uint32(0x8000)) >> jnp.uint32(16)
    return lax.bitcast_convert_type(hi | lo, jnp.int32)


def _unpack_hi(x_i32):
    xu = lax.bitcast_convert_type(x_i32, jnp.uint32)
    return lax.bitcast_convert_type(xu & jnp.uint32(0xFFFF0000), jnp.float32)


def _unpack_lo(x_i32):
    xu = lax.bitcast_convert_type(x_i32, jnp.uint32)
    return lax.bitcast_convert_type(xu << jnp.uint32(16), jnp.float32)


@functools.lru_cache(maxsize=None)
def _sc_gather_fn():
    mesh = plsc.VectorSubcoreMesh(core_axis_name="c", subcore_axis_name="s")

    def body(tab, ridx, cidx, hr, hc, idx_r, idx_c,
             br0, br1, bc0, bc1, sr0, sr1, sc0, sc1):
        wid = lax.axis_index("c") * NSUB + lax.axis_index("s")
        base = wid * EPW
        pltpu.sync_copy(ridx.at[wid], idx_r)
        pltpu.sync_copy(cidx.at[wid], idx_c)
        pltpu.async_copy(tab.at[idx_r.at[0]], br0, sr0)
        pltpu.async_copy(tab.at[idx_c.at[0]], bc0, sc0)
        pltpu.async_copy(tab.at[idx_r.at[1]], br1, sr1)
        pltpu.async_copy(tab.at[idx_c.at[1]], bc1, sc1)

        def lane(j, idx, buf, sem, dst):
            pltpu.make_async_copy(tab.at[idx.at[j]], buf, sem).wait()
            pltpu.sync_copy(buf, dst.at[pl.ds(base + j * CHUNK, CHUNK)])

            @pl.when(j + 2 < NCH)
            def _():
                pltpu.async_copy(tab.at[idx.at[j + 2]], buf, sem)

        def step(i, carry):
            j = 2 * i
            lane(j, idx_r, br0, sr0, hr)
            lane(j, idx_c, bc0, sc0, hc)
            lane(j + 1, idx_r, br1, sr1, hr)
            lane(j + 1, idx_c, bc1, sc1, hc)
            return carry

        lax.fori_loop(0, NCH // 2, step, 0)

    return pl.kernel(
        body,
        out_type=(jax.ShapeDtypeStruct((EPH, F), jnp.int32),
                  jax.ShapeDtypeStruct((EPH, F), jnp.int32)),
        mesh=mesh,
        scratch_types=[
            pltpu.VMEM((NCH, CHUNK), jnp.int32),
            pltpu.VMEM((NCH, CHUNK), jnp.int32),
            pltpu.VMEM((CHUNK, F), jnp.int32),
            pltpu.VMEM((CHUNK, F), jnp.int32),
            pltpu.VMEM((CHUNK, F), jnp.int32),
            pltpu.VMEM((CHUNK, F), jnp.int32),
            pltpu.SemaphoreType.DMA,
            pltpu.SemaphoreType.DMA,
            pltpu.SemaphoreType.DMA,
            pltpu.SemaphoreType.DMA,
        ],
    )


@functools.lru_cache(maxsize=None)
def _sc_scatter_fn(do_aux):
    mesh = plsc.VectorSubcoreMesh(core_axis_name="c", subcore_axis_name="s")

    def body(m0, m1, aux0, aux1, sidx0, sidx1, z128, aggm, agga,
             idx0, idx1, val0, val1, sem0, sem1, acc):
        c = lax.axis_index("c")
        s = lax.axis_index("s")
        wid = c * NSUB + s
        base = wid * EPW
        pltpu.sync_copy(sidx0.at[wid], idx0)
        pltpu.sync_copy(sidx1.at[wid], idx1)

        def half_loop(src, idx):
            pltpu.async_copy(src.at[pl.ds(base, CHUNK)], val0, sem0)

            def step(i, carry):
                j = 2 * i
                pltpu.async_copy(src.at[pl.ds(base + (j + 1) * CHUNK, CHUNK)],
                                 val1, sem1)
                pltpu.make_async_copy(src.at[pl.ds(base, CHUNK)], val0,
                                      sem0).wait()
                pltpu.sync_copy(val0, acc.at[idx.at[j]], add=True)

                @pl.when(j + 2 < NCH)
                def _():
                    pltpu.async_copy(
                        src.at[pl.ds(base + (j + 2) * CHUNK, CHUNK)],
                        val0, sem0)

                pltpu.make_async_copy(src.at[pl.ds(base, CHUNK)], val1,
                                      sem1).wait()
                pltpu.sync_copy(val1, acc.at[idx.at[j + 1]], add=True)
                return carry

            lax.fori_loop(0, NCH // 2, step, 0)

        def one_phase(srcs, dst):
            pltpu.sync_copy(z128, acc.at[pl.ds(s * ZPS, ZPS)])
            plsc.subcore_barrier()
            for src, idx in srcs:
                half_loop(src, idx)
            plsc.subcore_barrier()

            @pl.when(s == NSUB - 1)
            def _():
                pltpu.sync_copy(
                    acc.at[pl.ds((NSUB - 1) * OPS, N - (NSUB - 1) * OPS)],
                    dst.at[c].at[pl.ds((NSUB - 1) * OPS, N - (NSUB - 1) * OPS)])

            @pl.when(s < NSUB - 1)
            def _():
                pltpu.sync_copy(acc.at[pl.ds(s * OPS, OPS)],
                                dst.at[c].at[pl.ds(s * OPS, OPS)])

            plsc.subcore_barrier()

        one_phase(((m0, idx0), (m1, idx1)), aggm)
        if do_aux:
            one_phase(((aux0, idx0), (aux1, idx1)), agga)

    return pl.kernel(
        body,
        out_type=(jax.ShapeDtypeStruct((NC, N, F), jnp.float32),
                  jax.ShapeDtypeStruct((NC, N, F), jnp.float32)),
        mesh=mesh,
        scratch_types=[
            pltpu.VMEM((NCH, CHUNK), jnp.int32),
            pltpu.VMEM((NCH, CHUNK), jnp.int32),
            pltpu.VMEM((CHUNK, F), jnp.float32),
            pltpu.VMEM((CHUNK, F), jnp.float32),
            pltpu.SemaphoreType.DMA,
            pltpu.SemaphoreType.DMA,
            pltpu.VMEM_SHARED((ACC_ROWS, F), jnp.float32),
        ],
    )


def _edge_body(half_off, hr_ref, hc_ref, ea_ref, a1, b1w, e1, r1, bb1,
               a2, bb2, c1, bc1, c2, m_ref, aux_ref):
    hr32 = hr_ref[...]
    hc32 = hc_ref[...]
    d = _unpack_lo(hr32) - _unpack_lo(hc32)
    radial = jnp.sum(d * d, axis=1, keepdims=True)
    bf = jnp.bfloat16
    z = jnp.dot(_unpack_hi(hr32).astype(bf), a1[...],
                preferred_element_type=jnp.float32)
    z = z + jnp.dot(_unpack_hi(hc32).astype(bf), b1w[...],
                    preferred_element_type=jnp.float32)
    z = z + jnp.dot(ea_ref[...], e1[...], preferred_element_type=jnp.float32)
    z = z + radial * r1[...] + bb1[...]
    z = _silu(z)
    mm = _silu(jnp.dot(z.astype(bf), a2[...],
                       preferred_element_type=jnp.float32) + bb2[...])
    t = jnp.dot(_silu(jnp.dot(mm.astype(bf), c1[...],
                              preferred_element_type=jnp.float32)
                      + bc1[...]),
                c2[...], preferred_element_type=jnp.float32)
    one3 = (lax.broadcasted_iota(jnp.int32, (BE, F), 1) == 3).astype(jnp.float32)
    gid = (pl.program_id(0) * BE + half_off
           + lax.broadcasted_iota(jnp.int32, (BE, 1), 0))
    live = (gid < E).astype(jnp.float32)
    m_ref[...] = mm * live
    aux_ref[...] = (d * t + one3) * live


def _tc_edge(hr, hc, ea, half, w):
    full = lambda shp: pl.BlockSpec(shp, lambda i: (0,) * len(shp))
    nblk = EPH // BE
    return pl.pallas_call(
        functools.partial(_edge_body, half * EPH),
        grid=(nblk,),
        in_specs=[
            pl.BlockSpec((BE, F), lambda i: (i, 0)),
            pl.BlockSpec((BE, F), lambda i: (i, 0)),
            pl.BlockSpec((BE, ED), lambda i, h=half: (i + h * nblk, 0)),
            full((F, F)), full((F, F)), full((ED, F)), full((1, F)),
            full((1, F)), full((F, F)), full((1, F)), full((F, F)),
            full((1, F)), full((F, 1)),
        ],
        out_specs=[
            pl.BlockSpec((BE, F), lambda i: (i, 0)),
            pl.BlockSpec((BE, F), lambda i: (i, 0)),
        ],
        out_shape=[
            jax.ShapeDtypeStruct((EPH, F), jnp.float32),
            jax.ShapeDtypeStruct((EPH, F), jnp.float32),
        ],
    )(hr, hc, ea, *w)


def _node_body(hx_ref, h0x_ref, am_ref, aa_ref,
               n1h, n1a, n1n, nb1, n2, nb2, out_ref, tb_ref):
    hx = hx_ref[...]
    h = hx[:, 0:128]
    aggm = am_ref[0] + am_ref[1]
    agga = aa_ref[0] + aa_ref[1]
    cnt = jnp.maximum(agga[:, 3:4], 1.0)
    mask3 = (lax.broadcasted_iota(jnp.int32, (BN, F), 1) < 3).astype(jnp.float32)
    newc = hx[:, 128:256] + (agga / cnt) * mask3
    z = jnp.dot(h, n1h[...], preferred_element_type=jnp.float32)
    z = z + jnp.dot(aggm, n1a[...], preferred_element_type=jnp.float32)
    z = z + jnp.dot(h0x_ref[...][:, 0:128], n1n[...],
                    preferred_element_type=jnp.float32)
    z = _silu(z + nb1[...])
    hn = h + jnp.dot(z, n2[...], preferred_element_type=jnp.float32) + nb2[...]
    out_ref[:, 0:128] = hn
    out_ref[:, 128:256] = newc
    tb_ref[...] = _pack16(hn, newc)


def _tc_node(hx, h0x, am, aa, w):
    full = lambda shp: pl.BlockSpec(shp, lambda i: (0,) * len(shp))
    agg_spec = pl.BlockSpec((NC, BN, F), lambda i: (0, i, 0))
    return pl.pallas_call(
        _node_body,
        grid=(N // BN,),
        in_specs=[
            pl.BlockSpec((BN, DX), lambda i: (i, 0)),
            pl.BlockSpec((BN, DX), lambda i: (i, 0)),
            agg_spec, agg_spec,
            full((F, F)), full((F, F)), full((F, F)), full((1, F)),
            full((F, F)), full((1, F)),
        ],
        out_specs=[pl.BlockSpec((BN, DX), lambda i: (i, 0)),
                   pl.BlockSpec((BN, F), lambda i: (i, 0))],
        out_shape=[jax.ShapeDtypeStruct((N, DX), jnp.float32),
                   jax.ShapeDtypeStruct((N, F), jnp.int32)],
    )(hx, h0x, am, aa, *w)


def _node_pool_body(hx_ref, am_ref, n1h, n1a, nb1, n2, nb2,
                    out_ref):
    h = hx_ref[...][:, 0:128]
    aggm = am_ref[0] + am_ref[1]
    z = jnp.dot(h, n1h[...], preferred_element_type=jnp.float32)
    z = z + jnp.dot(aggm, n1a[...], preferred_element_type=jnp.float32)
    z = _silu(z + nb1[...])
    out_ref[...] = h + jnp.dot(z, n2[...], preferred_element_type=jnp.float32) \
        + nb2[...]


def _tc_node_pool(hx, am, w):
    full = lambda shp: pl.BlockSpec(shp, lambda i: (0,) * len(shp))
    agg_spec = pl.BlockSpec((NC, BN, F), lambda i: (0, i, 0))
    return pl.pallas_call(
        _node_pool_body,
        grid=(N // BN,),
        in_specs=[
            pl.BlockSpec((BN, DX), lambda i: (i, 0)),
            agg_spec,
            full((F, F)), full((F, F)), full((1, F)), full((F, F)),
            full((1, F)),
        ],
        out_specs=pl.BlockSpec((BN, F), lambda i: (i, 0)),
        out_shape=jax.ShapeDtypeStruct((N, F), jnp.float32),
    )(hx, am, *w)


def _embed_body(hx_ref, wt, bb, out_ref, tb_ref):
    hx = hx_ref[...]
    he = jnp.dot(hx[:, 0:128], wt[...],
                 preferred_element_type=jnp.float32) + bb[...]
    out_ref[:, 0:128] = he
    out_ref[:, 128:256] = hx[:, 128:256]
    tb_ref[...] = _pack16(he, hx[:, 128:256])


def _tc_embed(hx, wt, bb):
    full = lambda shp: pl.BlockSpec(shp, lambda i: (0,) * len(shp))
    return pl.pallas_call(
        _embed_body,
        grid=(N // BN,),
        in_specs=[
            pl.BlockSpec((BN, DX), lambda i: (i, 0)),
            full((F, F)), full((1, F)),
        ],
        out_specs=[pl.BlockSpec((BN, DX), lambda i: (i, 0)),
                   pl.BlockSpec((BN, F), lambda i: (i, 0))],
        out_shape=[jax.ShapeDtypeStruct((N, DX), jnp.float32),
                   jax.ShapeDtypeStruct((N, F), jnp.int32)],
    )(hx, wt, bb)


def _head_body(hf_ref, batch_ref, ti_ref, te_ref,
               m0h, m0t, hb0, m1, hb1, m2, hb2, m3, hb3, out_ref):
    hf = hf_ref[...]
    oneh = (lax.broadcasted_iota(jnp.int32, (NB, N), 0)
            == batch_ref[...]).astype(jnp.float32)
    sums = jnp.dot(oneh, hf, preferred_element_type=jnp.float32)
    cnt = jnp.maximum(jnp.sum(oneh, axis=1, keepdims=True), 1.0)
    p = sums / cnt
    teh = (lax.broadcasted_iota(jnp.int32, (NB, NTASK), 1)
           == ti_ref[...]).astype(jnp.float32)
    te = jnp.dot(teh, te_ref[...], preferred_element_type=jnp.float32)
    y = jnp.maximum(jnp.dot(p, m0h[...], preferred_element_type=jnp.float32)
                    + jnp.dot(te, m0t[...], preferred_element_type=jnp.float32)
                    + hb0[...], 0.0)
    y = jnp.maximum(jnp.dot(y, m1[...], preferred_element_type=jnp.float32)
                    + hb1[...], 0.0)
    y = jnp.maximum(jnp.dot(y, m2[...], preferred_element_type=jnp.float32)
                    + hb2[...], 0.0)
    out_ref[...] = jnp.dot(y, m3[...], preferred_element_type=jnp.float32) \
        + hb3[...]


def _tc_head(hf, batch_row, ti, te, w):
    full = lambda shp: pl.BlockSpec(shp, lambda: (0,) * len(shp))
    return pl.pallas_call(
        _head_body,
        in_specs=[
            full((N, F)), full((1, N)), full((NB, 1)), full((NTASK, TED)),
            full((F, F)), full((TED, F)), full((1, F)),
            full((F, F)), full((1, F)),
            full((F, F)), full((1, F)),
            full((F, NCLS)), full((1, NCLS)),
        ],
        out_specs=full((NB, NCLS)),
        out_shape=jax.ShapeDtypeStruct((NB, NCLS), jnp.float32),
    )(hf, batch_row, ti, te, *w)


def _edge_w(p):
    w1 = p["edge1"]["W"]
    bf = jnp.bfloat16
    return (w1[:, 0:128].T.astype(bf), w1[:, 128:256].T.astype(bf),
            w1[:, 257:273].T,
            w1[:, 256:257].T, p["edge1"]["b"][None],
            p["edge2"]["W"].T.astype(bf), p["edge2"]["b"][None],
            p["coord1"]["W"].T.astype(bf), p["coord1"]["b"][None],
            p["coord2"]["W"].T)


def _node_w(p):
    wn = p["node1"]["W"]
    return (wn[:, 0:128].T, wn[:, 128:256].T, wn[:, 256:384].T,
            p["node1"]["b"][None], p["node2"]["W"].T, p["node2"]["b"][None])


def _node_pool_w(p):
    wn = p["node1"]["W"]
    return (wn[:, 0:128].T, wn[:, 128:256].T,
            p["node1"]["b"][None], p["node2"]["W"].T, p["node2"]["b"][None])


def _head_w(params):
    w0 = params["mlp0"]["W"]
    return (w0[:, 0:128].T, w0[:, 128:192].T, params["mlp0"]["b"][None],
            params["mlp1"]["W"].T, params["mlp1"]["b"][None],
            params["mlp2"]["W"].T, params["mlp2"]["b"][None],
            params["mlp3"]["W"].T, params["mlp3"]["b"][None])


def kernel(h, x, edge_index, edge_attr, batch, tasks_indices, params):
    f32 = jnp.float32
    row = edge_index[0].astype(jnp.int32)
    col = edge_index[1].astype(jnp.int32)
    npad = EPAD - E
    rowp = jnp.concatenate([row, row[:npad]])
    colp = jnp.concatenate([col, col[:npad]])
    rowg = rowp.reshape(NHALF, NW, NCH, CHUNK)
    colg = colp.reshape(NHALF, NW, NCH, CHUNK)
    rows = rowg
    ea = jnp.zeros((EPAD, ED), f32).at[:E].set(edge_attr.astype(f32))
    z128 = jnp.zeros((ZPS, F), f32)

    xpad = jnp.zeros((N, F), f32).at[:, 0:3].set(x.astype(f32))
    hx = jnp.concatenate([h.astype(f32), xpad], axis=1)

    gather = _sc_gather_fn()

    def egcl(hx_cur, tb32, h0x, p, pool):
        scatter = _sc_scatter_fn(not pool)
        ew = _edge_w(p)
        ms, auxs = [], []
        for hh in range(NHALF):
            hr, hc = gather(tb32, rowg[hh], colg[hh])
            m, aux = _tc_edge(hr, hc, ea, hh, ew)
            ms.append(m)
            auxs.append(aux)
        aggm, agga = scatter(ms[0], ms[1], auxs[0], auxs[1],
                             rows[0], rows[1], z128)
        if pool:
            return _tc_node_pool(hx_cur, aggm, _node_pool_w(p))
        return _tc_node(hx_cur, h0x, aggm, agga, _node_w(p))

    tb32 = None
    for ep in params["egnns"]:
        h0x = hx
        hx, tb32 = _tc_embed(hx, ep["embed"]["W"].T, ep["embed"]["b"][None])
        for lp in ep["layers"]:
            hx, tb32 = egcl(hx, tb32, h0x, lp, pool=False)

    hfin = egcl(hx, tb32, None, params["pool_egcl"], pool=True)

    batch_row = batch.astype(jnp.int32).reshape(1, N)
    ti = tasks_indices[:, 1].astype(jnp.int32).reshape(NB, 1)
    return _tc_head(hfin, batch_row, ti, params["tasks_embed"].astype(f32),
                    _head_w(params))


# speedup vs baseline: 1.9312x; 1.0409x over previous
import functools

import jax
import jax.numpy as jnp
from jax import lax
from jax.experimental import pallas as pl
from jax.experimental.pallas import tpu as pltpu
from jax.experimental.pallas import tpu_sc as plsc

N = 10000
E = 160000
F = 128
ED = 16
DX = 256
NB = 16
NTASK = 64
TED = 64
NCLS = 3

NC, NSUB = 2, 16
NW = NC * NSUB
CHUNK = 128
NCH = 20
EPW = CHUNK * NCH
EPH = NW * EPW
NHALF = 2
EPAD = NHALF * EPH
ACC_ROWS = 10240
ZPS = ACC_ROWS // NSUB
OPS = 624
DUMMY = N

BE = 2048
BN = 1000


def _silu(v):
    return v * (1.0 / (1.0 + jnp.exp(-v)))


def _pack16(hi_f32, lo_f32):
    hb = lax.bitcast_convert_type(hi_f32, jnp.uint32)
    lb = lax.bitcast_convert_type(lo_f32, jnp.uint32)
    hi = (hb + jnp.uint32(0x8000)) & jnp.uint32(0xFFFF0000)
    lo = (lb + jnp.uint32(0x8000)) >> jnp.uint32(16)
    return lax.bitcast_convert_type(hi | lo, jnp.int32)


def _unpack_hi(x_i32):
    xu = lax.bitcast_convert_type(x_i32, jnp.uint32)
    return lax.bitcast_convert_type(xu & jnp.uint32(0xFFFF0000), jnp.float32)


def _unpack_lo(x_i32):
    xu = lax.bitcast_convert_type(x_i32, jnp.uint32)
    return lax.bitcast_convert_type(xu << jnp.uint32(16), jnp.float32)


@functools.lru_cache(maxsize=None)
def _sc_gather_fn():
    mesh = plsc.VectorSubcoreMesh(core_axis_name="c", subcore_axis_name="s")

    def body(tab, ridx, cidx, hr, hc, idx_r, idx_c,
             br0, br1, bc0, bc1, sr0, sr1, sc0, sc1):
        wid = lax.axis_index("c") * NSUB + lax.axis_index("s")
        base = wid * EPW
        pltpu.sync_copy(ridx.at[wid], idx_r)
        pltpu.sync_copy(cidx.at[wid], idx_c)
        pltpu.async_copy(tab.at[idx_r.at[0]], br0, sr0)
        pltpu.async_copy(tab.at[idx_c.at[0]], bc0, sc0)
        pltpu.async_copy(tab.at[idx_r.at[1]], br1, sr1)
        pltpu.async_copy(tab.at[idx_c.at[1]], bc1, sc1)

        def lane(j, idx, buf, sem, dst):
            pltpu.make_async_copy(tab.at[idx.at[j]], buf, sem).wait()
            pltpu.sync_copy(buf, dst.at[pl.ds(base + j * CHUNK, CHUNK)])

            @pl.when(j + 2 < NCH)
            def _():
                pltpu.async_copy(tab.at[idx.at[j + 2]], buf, sem)

        def step(i, carry):
            j = 2 * i
            lane(j, idx_r, br0, sr0, hr)
            lane(j, idx_c, bc0, sc0, hc)
            lane(j + 1, idx_r, br1, sr1, hr)
            lane(j + 1, idx_c, bc1, sc1, hc)
            return carry

        lax.fori_loop(0, NCH // 2, step, 0)

    return pl.kernel(
        body,
        out_type=(jax.ShapeDtypeStruct((EPH, F), jnp.int32),
                  jax.ShapeDtypeStruct((EPH, F), jnp.int32)),
        mesh=mesh,
        scratch_types=[
            pltpu.VMEM((NCH, CHUNK), jnp.int32),
            pltpu.VMEM((NCH, CHUNK), jnp.int32),
            pltpu.VMEM((CHUNK, F), jnp.int32),
            pltpu.VMEM((CHUNK, F), jnp.int32),
            pltpu.VMEM((CHUNK, F), jnp.int32),
            pltpu.VMEM((CHUNK, F), jnp.int32),
            pltpu.SemaphoreType.DMA,
            pltpu.SemaphoreType.DMA,
            pltpu.SemaphoreType.DMA,
            pltpu.SemaphoreType.DMA,
        ],
    )


@functools.lru_cache(maxsize=None)
def _sc_scatter_fn(do_aux):
    mesh = plsc.VectorSubcoreMesh(core_axis_name="c", subcore_axis_name="s")

    def body(m0, aux0, sidx0, z128, aggm, agga,
             idx0, val0, val1, sem0, sem1, acc):
        c = lax.axis_index("c")
        s = lax.axis_index("s")
        wid = c * NSUB + s
        base = wid * EPW
        pltpu.sync_copy(sidx0.at[wid], idx0)

        def half_loop(src, idx):
            pltpu.async_copy(src.at[pl.ds(base, CHUNK)], val0, sem0)

            def step(i, carry):
                j = 2 * i
                pltpu.async_copy(src.at[pl.ds(base + (j + 1) * CHUNK, CHUNK)],
                                 val1, sem1)
                pltpu.make_async_copy(src.at[pl.ds(base, CHUNK)], val0,
                                      sem0).wait()
                pltpu.sync_copy(val0, acc.at[idx.at[j]], add=True)

                @pl.when(j + 2 < NCH)
                def _():
                    pltpu.async_copy(
                        src.at[pl.ds(base + (j + 2) * CHUNK, CHUNK)],
                        val0, sem0)

                pltpu.make_async_copy(src.at[pl.ds(base, CHUNK)], val1,
                                      sem1).wait()
                pltpu.sync_copy(val1, acc.at[idx.at[j + 1]], add=True)
                return carry

            lax.fori_loop(0, NCH // 2, step, 0)

        def one_phase(srcs, dst):
            pltpu.sync_copy(z128, acc.at[pl.ds(s * ZPS, ZPS)])
            plsc.subcore_barrier()
            for src, idx in srcs:
                half_loop(src, idx)
            plsc.subcore_barrier()

            @pl.when(s == NSUB - 1)
            def _():
                pltpu.sync_copy(
                    acc.at[pl.ds((NSUB - 1) * OPS, N - (NSUB - 1) * OPS)],
                    dst.at[c].at[pl.ds((NSUB - 1) * OPS, N - (NSUB - 1) * OPS)])

            @pl.when(s < NSUB - 1)
            def _():
                pltpu.sync_copy(acc.at[pl.ds(s * OPS, OPS)],
                                dst.at[c].at[pl.ds(s * OPS, OPS)])

            plsc.subcore_barrier()

        one_phase(((m0, idx0),), aggm)
        if do_aux:
            one_phase(((aux0, idx0),), agga)

    return pl.kernel(
        body,
        out_type=(jax.ShapeDtypeStruct((NC, N, F), jnp.float32),
                  jax.ShapeDtypeStruct((NC, N, F), jnp.float32)),
        mesh=mesh,
        scratch_types=[
            pltpu.VMEM((NCH, CHUNK), jnp.int32),
            pltpu.VMEM((CHUNK, F), jnp.float32),
            pltpu.VMEM((CHUNK, F), jnp.float32),
            pltpu.SemaphoreType.DMA,
            pltpu.SemaphoreType.DMA,
            pltpu.VMEM_SHARED((ACC_ROWS, F), jnp.float32),
        ],
    )


def _edge_body(half_off, hr_ref, hc_ref, ea_ref, a1, b1w, e1, r1, bb1,
               a2, bb2, c1, bc1, c2, m_ref, aux_ref):
    hr32 = hr_ref[...]
    hc32 = hc_ref[...]
    d = _unpack_lo(hr32) - _unpack_lo(hc32)
    radial = jnp.sum(d * d, axis=1, keepdims=True)
    bf = jnp.bfloat16
    z = jnp.dot(_unpack_hi(hr32).astype(bf), a1[...],
                preferred_element_type=jnp.float32)
    z = z + jnp.dot(_unpack_hi(hc32).astype(bf), b1w[...],
                    preferred_element_type=jnp.float32)
    z = z + jnp.dot(ea_ref[...], e1[...], preferred_element_type=jnp.float32)
    z = z + radial * r1[...] + bb1[...]
    z = _silu(z)
    mm = _silu(jnp.dot(z.astype(bf), a2[...],
                       preferred_element_type=jnp.float32) + bb2[...])
    t = jnp.dot(_silu(jnp.dot(mm.astype(bf), c1[...],
                              preferred_element_type=jnp.float32)
                      + bc1[...]),
                c2[...], preferred_element_type=jnp.float32)
    one3 = (lax.broadcasted_iota(jnp.int32, (BE, F), 1) == 3).astype(jnp.float32)
    gid = (pl.program_id(0) * BE + half_off
           + lax.broadcasted_iota(jnp.int32, (BE, 1), 0))
    live = (gid < E).astype(jnp.float32)
    m_ref[...] = mm * live
    aux_ref[...] = (d * t + one3) * live


def _tc_edge(hr, hc, ea, half, w):
    full = lambda shp: pl.BlockSpec(shp, lambda i: (0,) * len(shp))
    nblk = EPH // BE
    return pl.pallas_call(
        functools.partial(_edge_body, half * EPH),
        grid=(nblk,),
        in_specs=[
            pl.BlockSpec((BE, F), lambda i: (i, 0)),
            pl.BlockSpec((BE, F), lambda i: (i, 0)),
            pl.BlockSpec((BE, ED), lambda i, h=half: (i + h * nblk, 0)),
            full((F, F)), full((F, F)), full((ED, F)), full((1, F)),
            full((1, F)), full((F, F)), full((1, F)), full((F, F)),
            full((1, F)), full((F, 1)),
        ],
        out_specs=[
            pl.BlockSpec((BE, F), lambda i: (i, 0)),
            pl.BlockSpec((BE, F), lambda i: (i, 0)),
        ],
        out_shape=[
            jax.ShapeDtypeStruct((EPH, F), jnp.float32),
            jax.ShapeDtypeStruct((EPH, F), jnp.float32),
        ],
    )(hr, hc, ea, *w)


def _node_body(hx_ref, h0x_ref, am_ref, am1_ref, aa_ref, aa1_ref,
               n1h, n1a, n1n, nb1, n2, nb2, out_ref, tb_ref):
    hx = hx_ref[...]
    h = hx[:, 0:128]
    aggm = am_ref[0] + am_ref[1] + am1_ref[0] + am1_ref[1]
    agga = aa_ref[0] + aa_ref[1] + aa1_ref[0] + aa1_ref[1]
    cnt = jnp.maximum(agga[:, 3:4], 1.0)
    mask3 = (lax.broadcasted_iota(jnp.int32, (BN, F), 1) < 3).astype(jnp.float32)
    newc = hx[:, 128:256] + (agga / cnt) * mask3
    z = jnp.dot(h, n1h[...], preferred_element_type=jnp.float32)
    z = z + jnp.dot(aggm, n1a[...], preferred_element_type=jnp.float32)
    z = z + jnp.dot(h0x_ref[...][:, 0:128], n1n[...],
                    preferred_element_type=jnp.float32)
    z = _silu(z + nb1[...])
    hn = h + jnp.dot(z, n2[...], preferred_element_type=jnp.float32) + nb2[...]
    out_ref[:, 0:128] = hn
    out_ref[:, 128:256] = newc
    tb_ref[...] = _pack16(hn, newc)


def _tc_node(hx, h0x, am, am1, aa, aa1, w):
    full = lambda shp: pl.BlockSpec(shp, lambda i: (0,) * len(shp))
    agg_spec = pl.BlockSpec((NC, BN, F), lambda i: (0, i, 0))
    return pl.pallas_call(
        _node_body,
        grid=(N // BN,),
        in_specs=[
            pl.BlockSpec((BN, DX), lambda i: (i, 0)),
            pl.BlockSpec((BN, DX), lambda i: (i, 0)),
            agg_spec, agg_spec, agg_spec, agg_spec,
            full((F, F)), full((F, F)), full((F, F)), full((1, F)),
            full((F, F)), full((1, F)),
        ],
        out_specs=[pl.BlockSpec((BN, DX), lambda i: (i, 0)),
                   pl.BlockSpec((BN, F), lambda i: (i, 0))],
        out_shape=[jax.ShapeDtypeStruct((N, DX), jnp.float32),
                   jax.ShapeDtypeStruct((N, F), jnp.int32)],
    )(hx, h0x, am, am1, aa, aa1, *w)


def _node_pool_body(hx_ref, am_ref, am1_ref, n1h, n1a, nb1, n2, nb2,
                    out_ref):
    h = hx_ref[...][:, 0:128]
    aggm = am_ref[0] + am_ref[1] + am1_ref[0] + am1_ref[1]
    z = jnp.dot(h, n1h[...], preferred_element_type=jnp.float32)
    z = z + jnp.dot(aggm, n1a[...], preferred_element_type=jnp.float32)
    z = _silu(z + nb1[...])
    out_ref[...] = h + jnp.dot(z, n2[...], preferred_element_type=jnp.float32) \
        + nb2[...]


def _tc_node_pool(hx, am, am1, w):
    full = lambda shp: pl.BlockSpec(shp, lambda i: (0,) * len(shp))
    agg_spec = pl.BlockSpec((NC, BN, F), lambda i: (0, i, 0))
    return pl.pallas_call(
        _node_pool_body,
        grid=(N // BN,),
        in_specs=[
            pl.BlockSpec((BN, DX), lambda i: (i, 0)),
            agg_spec, agg_spec,
            full((F, F)), full((F, F)), full((1, F)), full((F, F)),
            full((1, F)),
        ],
        out_specs=pl.BlockSpec((BN, F), lambda i: (i, 0)),
        out_shape=jax.ShapeDtypeStruct((N, F), jnp.float32),
    )(hx, am, am1, *w)


def _embed_body(hx_ref, wt, bb, out_ref, tb_ref):
    hx = hx_ref[...]
    he = jnp.dot(hx[:, 0:128], wt[...],
                 preferred_element_type=jnp.float32) + bb[...]
    out_ref[:, 0:128] = he
    out_ref[:, 128:256] = hx[:, 128:256]
    tb_ref[...] = _pack16(he, hx[:, 128:256])


def _tc_embed(hx, wt, bb):
    full = lambda shp: pl.BlockSpec(shp, lambda i: (0,) * len(shp))
    return pl.pallas_call(
        _embed_body,
        grid=(N // BN,),
        in_specs=[
            pl.BlockSpec((BN, DX), lambda i: (i, 0)),
            full((F, F)), full((1, F)),
        ],
        out_specs=[pl.BlockSpec((BN, DX), lambda i: (i, 0)),
                   pl.BlockSpec((BN, F), lambda i: (i, 0))],
        out_shape=[jax.ShapeDtypeStruct((N, DX), jnp.float32),
                   jax.ShapeDtypeStruct((N, F), jnp.int32)],
    )(hx, wt, bb)


def _head_body(hf_ref, batch_ref, ti_ref, te_ref,
               m0h, m0t, hb0, m1, hb1, m2, hb2, m3, hb3, out_ref):
    hf = hf_ref[...]
    oneh = (lax.broadcasted_iota(jnp.int32, (NB, N), 0)
            == batch_ref[...]).astype(jnp.float32)
    sums = jnp.dot(oneh, hf, preferred_element_type=jnp.float32)
    cnt = jnp.maximum(jnp.sum(oneh, axis=1, keepdims=True), 1.0)
    p = sums / cnt
    teh = (lax.broadcasted_iota(jnp.int32, (NB, NTASK), 1)
           == ti_ref[...]).astype(jnp.float32)
    te = jnp.dot(teh, te_ref[...], preferred_element_type=jnp.float32)
    y = jnp.maximum(jnp.dot(p, m0h[...], preferred_element_type=jnp.float32)
                    + jnp.dot(te, m0t[...], preferred_element_type=jnp.float32)
                    + hb0[...], 0.0)
    y = jnp.maximum(jnp.dot(y, m1[...], preferred_element_type=jnp.float32)
                    + hb1[...], 0.0)
    y = jnp.maximum(jnp.dot(y, m2[...], preferred_element_type=jnp.float32)
                    + hb2[...], 0.0)
    out_ref[...] = jnp.dot(y, m3[...], preferred_element_type=jnp.float32) \
        + hb3[...]


def _tc_head(hf, batch_row, ti, te, w):
    full = lambda shp: pl.BlockSpec(shp, lambda: (0,) * len(shp))
    return pl.pallas_call(
        _head_body,
        in_specs=[
            full((N, F)), full((1, N)), full((NB, 1)), full((NTASK, TED)),
            full((F, F)), full((TED, F)), full((1, F)),
            full((F, F)), full((1, F)),
            full((F, F)), full((1, F)),
            full((F, NCLS)), full((1, NCLS)),
        ],
        out_specs=full((NB, NCLS)),
        out_shape=jax.ShapeDtypeStruct((NB, NCLS), jnp.float32),
    )(hf, batch_row, ti, te, *w)


def _edge_w(p):
    w1 = p["edge1"]["W"]
    bf = jnp.bfloat16
    return (w1[:, 0:128].T.astype(bf), w1[:, 128:256].T.astype(bf),
            w1[:, 257:273].T,
            w1[:, 256:257].T, p["edge1"]["b"][None],
            p["edge2"]["W"].T.astype(bf), p["edge2"]["b"][None],
            p["coord1"]["W"].T.astype(bf), p["coord1"]["b"][None],
            p["coord2"]["W"].T)


def _node_w(p):
    wn = p["node1"]["W"]
    return (wn[:, 0:128].T, wn[:, 128:256].T, wn[:, 256:384].T,
            p["node1"]["b"][None], p["node2"]["W"].T, p["node2"]["b"][None])


def _node_pool_w(p):
    wn = p["node1"]["W"]
    return (wn[:, 0:128].T, wn[:, 128:256].T,
            p["node1"]["b"][None], p["node2"]["W"].T, p["node2"]["b"][None])


def _head_w(params):
    w0 = params["mlp0"]["W"]
    return (w0[:, 0:128].T, w0[:, 128:192].T, params["mlp0"]["b"][None],
            params["mlp1"]["W"].T, params["mlp1"]["b"][None],
            params["mlp2"]["W"].T, params["mlp2"]["b"][None],
            params["mlp3"]["W"].T, params["mlp3"]["b"][None])


def kernel(h, x, edge_index, edge_attr, batch, tasks_indices, params):
    f32 = jnp.float32
    row = edge_index[0].astype(jnp.int32)
    col = edge_index[1].astype(jnp.int32)
    npad = EPAD - E
    rowp = jnp.concatenate([row, row[:npad]])
    colp = jnp.concatenate([col, col[:npad]])
    rowg = rowp.reshape(NHALF, NW, NCH, CHUNK)
    colg = colp.reshape(NHALF, NW, NCH, CHUNK)
    rows = rowg
    ea = jnp.zeros((EPAD, ED), f32).at[:E].set(edge_attr.astype(f32))
    z128 = jnp.zeros((ZPS, F), f32)

    xpad = jnp.zeros((N, F), f32).at[:, 0:3].set(x.astype(f32))
    hx = jnp.concatenate([h.astype(f32), xpad], axis=1)

    gather = _sc_gather_fn()

    def egcl(hx_cur, tb32, h0x, p, pool):
        scatter = _sc_scatter_fn(not pool)
        ew = _edge_w(p)
        am, aa = [], []
        for hh in range(NHALF):
            hr, hc = gather(tb32, rowg[hh], colg[hh])
            m, aux = _tc_edge(hr, hc, ea, hh, ew)
            aggm, agga = scatter(m, aux, rows[hh], z128)
            am.append(aggm)
            aa.append(agga)
        if pool:
            return _tc_node_pool(hx_cur, am[0], am[1], _node_pool_w(p))
        return _tc_node(hx_cur, h0x, am[0], am[1], aa[0], aa[1], _node_w(p))

    tb32 = None
    for ep in params["egnns"]:
        h0x = hx
        hx, tb32 = _tc_embed(hx, ep["embed"]["W"].T, ep["embed"]["b"][None])
        for lp in ep["layers"]:
            hx, tb32 = egcl(hx, tb32, h0x, lp, pool=False)

    hfin = egcl(hx, tb32, None, params["pool_egcl"], pool=True)

    batch_row = batch.astype(jnp.int32).reshape(1, N)
    ti = tasks_indices[:, 1].astype(jnp.int32).reshape(NB, 1)
    return _tc_head(hfin, batch_row, ti, params["tasks_embed"].astype(f32),
                    _head_w(params))
